# Initial kernel scaffold; baseline (speedup 1.0000x reference)
import jax
import jax.numpy as jnp
from jax.experimental import pallas as pl


def kernel(x, edge_index, x1, edge_weight, W1_rel, b1_rel, W1_root, W2_rel, b2_rel, W2_root, W_lin, b_lin):
    raise NotImplementedError("write your pallas kernel here")


# speedup vs baseline: 5.5132x; 5.5132x over previous
import jax
import jax.numpy as jnp
from jax import lax
from jax.experimental import pallas as pl
from jax.experimental.pallas import tpu as pltpu
from jax.experimental.pallas import tpu_sc as plsc

_N = 10000
_D = 128
_E = 320000
_NP = 10240
_NC = 2
_NS = 16
_NW = _NC * _NS
_G = 128
_CH = 1024
_EP = 327680
_EW = _EP // _NW
_T = _EW // _CH
_RZ = _NP // _NS


def _make_segsum(width):
    mesh = plsc.VectorSubcoreMesh(core_axis_name="c", subcore_axis_name="s")
    ch = 256 if width == 128 else _CH
    tch = _EW // ch
    gpc = ch // _G

    def body(y, src2, dst2, w, out, accum,
             src_a, src_b, dst_a, dst_b, w_a, w_b, rows_a, rows_b,
             zb_v, gsem, ssem):
        c = lax.axis_index("c")
        s = lax.axis_index("s")
        wid = s * _NC + c
        srcb = (src_a, src_b)
        dstb = (dst_a, dst_b)
        wbuf = (w_a, w_b)
        rbuf = (rows_a, rows_b)

        for i in range(16):
            for hh in range(width // 16):
                zb_v[i, pl.ds(hh * 16, 16)] = jnp.zeros((16,), jnp.float32)

        def zloop(k, carry):
            pltpu.sync_copy(zb_v, accum.at[pl.ds(s * _RZ + k * 16, 16)])
            return carry

        lax.fori_loop(0, _RZ // 16, zloop, 0)
        plsc.subcore_barrier()

        def issue_gathers(t, b):
            g0 = wid * (_EW // _G) + t * gpc
            e0 = wid * _EW + t * ch
            pltpu.sync_copy(src2.at[pl.ds(g0, gpc)], srcb[b])
            pltpu.sync_copy(dst2.at[pl.ds(g0, gpc)], dstb[b])
            pltpu.sync_copy(w.at[pl.ds(e0, ch)], wbuf[b])
            return [
                pltpu.async_copy(y.at[srcb[b].at[j]],
                                 rbuf[b].at[pl.ds(j * _G, _G)], gsem)
                for j in range(gpc)
            ]

        pend_g = {0: issue_gathers(0, 0)}
        pend_s = {}
        for t in range(tch):
            b = t % 2
            for cp in pend_g.pop(t):
                cp.wait()
            if t + 1 < tch:
                for cp in pend_s.pop(t - 1, ()):
                    cp.wait()
                pend_g[t + 1] = issue_gathers(t + 1, 1 - b)

            def edge16(i, icarry, _b=b):
                wv16 = wbuf[_b][pl.ds(i * 16, 16)]
                for k in range(16):
                    e = i * 16 + k
                    wgt = wv16[k]
                    for hh in range(width // 16):
                        sl = pl.ds(hh * 16, 16)
                        rbuf[_b][e, sl] = rbuf[_b][e, sl] * wgt
                return icarry

            lax.fori_loop(0, ch // 16, edge16, 0)
            pend_s[t] = [
                pltpu.async_copy(rbuf[b].at[pl.ds(j * _G, _G)],
                                 accum.at[dstb[b].at[j]], ssem, add=True)
                for j in range(gpc)
            ]
        for t in sorted(pend_s):
            for cp in pend_s[t]:
                cp.wait()
        plsc.subcore_barrier()
        pltpu.sync_copy(accum.at[pl.ds(s * _RZ, _RZ)],
                        out.at[pl.ds(c * _NP + s * _RZ, _RZ)])

    return pl.kernel(
        body,
        out_type=jax.ShapeDtypeStruct((2 * _NP, width), jnp.float32),
        mesh=mesh,
        compiler_params=pltpu.CompilerParams(use_tc_tiling_on_sc=False),
        scratch_types=[
            pltpu.VMEM_SHARED((_NP, width), jnp.float32),
            pltpu.VMEM((gpc, _G), jnp.int32),
            pltpu.VMEM((gpc, _G), jnp.int32),
            pltpu.VMEM((gpc, _G), jnp.int32),
            pltpu.VMEM((gpc, _G), jnp.int32),
            pltpu.VMEM((ch,), jnp.float32),
            pltpu.VMEM((ch,), jnp.float32),
            pltpu.VMEM((ch, width), jnp.float32),
            pltpu.VMEM((ch, width), jnp.float32),
            pltpu.VMEM((16, width), jnp.float32),
            pltpu.SemaphoreType.DMA,
            pltpu.SemaphoreType.DMA,
        ],
    )


def _make_segsum_cols():
    mesh = plsc.VectorSubcoreMesh(core_axis_name="c", subcore_axis_name="s")
    wd = 64
    ch = 512
    ews = _EP // _NS
    tch = ews // ch
    gpc = ch // _G

    def body(xc, src2, dst2, w, out, accum,
             src_a, src_b, dst_a, dst_b, w_a, w_b, rows_a, rows_b,
             zb_v, gsem, ssem):
        c = lax.axis_index("c")
        s = lax.axis_index("s")
        coff = c * _NP
        srcb = (src_a, src_b)
        dstb = (dst_a, dst_b)
        wbuf = (w_a, w_b)
        rbuf = (rows_a, rows_b)

        for i in range(16):
            for hh in range(wd // 16):
                zb_v[i, pl.ds(hh * 16, 16)] = jnp.zeros((16,), jnp.float32)

        def zloop(k, carry):
            pltpu.sync_copy(zb_v, accum.at[pl.ds(s * _RZ + k * 16, 16)])
            return carry

        lax.fori_loop(0, _RZ // 16, zloop, 0)
        plsc.subcore_barrier()

        def issue_gathers(t, b):
            g0 = s * (ews // _G) + t * gpc
            e0 = s * ews + t * ch
            pltpu.sync_copy(src2.at[pl.ds(g0, gpc)], srcb[b])
            pltpu.sync_copy(dst2.at[pl.ds(g0, gpc)], dstb[b])
            pltpu.sync_copy(w.at[pl.ds(e0, ch)], wbuf[b])
            for j in range(gpc):
                for q in range(_G // 16):
                    sl = pl.ds(q * 16, 16)
                    srcb[b][j, sl] = srcb[b][j, sl] + coff
            return [
                pltpu.async_copy(xc.at[srcb[b].at[j]],
                                 rbuf[b].at[pl.ds(j * _G, _G)], gsem)
                for j in range(gpc)
            ]

        def mul_and_scatter(b):
            def edge16(i, icarry):
                wv16 = wbuf[b][pl.ds(i * 16, 16)]
                for k in range(16):
                    e = i * 16 + k
                    wgt = wv16[k]
                    for hh in range(wd // 16):
                        sl = pl.ds(hh * 16, 16)
                        rbuf[b][e, sl] = rbuf[b][e, sl] * wgt
                return icarry

            lax.fori_loop(0, ch // 16, edge16, 0)
            for j in range(gpc):
                pltpu.sync_copy(rbuf[b].at[pl.ds(j * _G, _G)],
                                accum.at[dstb[b].at[j]], add=True)

        def pair(i, carry):
            g0 = issue_gathers(2 * i, 0)
            g1 = issue_gathers(2 * i + 1, 1)
            for cp in g0:
                cp.wait()
            mul_and_scatter(0)
            for cp in g1:
                cp.wait()
            mul_and_scatter(1)
            return carry

        lax.fori_loop(0, tch // 2, pair, 0)
        plsc.subcore_barrier()
        pltpu.sync_copy(accum.at[pl.ds(s * _RZ, _RZ)],
                        out.at[pl.ds(c * _NP + s * _RZ, _RZ)])

    return pl.kernel(
        body,
        out_type=jax.ShapeDtypeStruct((2 * _NP, wd), jnp.float32),
        mesh=mesh,
        compiler_params=pltpu.CompilerParams(use_tc_tiling_on_sc=False),
        scratch_types=[
            pltpu.VMEM_SHARED((_NP, wd), jnp.float32),
            pltpu.VMEM((gpc, _G), jnp.int32),
            pltpu.VMEM((gpc, _G), jnp.int32),
            pltpu.VMEM((gpc, _G), jnp.int32),
            pltpu.VMEM((gpc, _G), jnp.int32),
            pltpu.VMEM((ch,), jnp.float32),
            pltpu.VMEM((ch,), jnp.float32),
            pltpu.VMEM((ch, wd), jnp.float32),
            pltpu.VMEM((ch, wd), jnp.float32),
            pltpu.VMEM((16, wd), jnp.float32),
            pltpu.SemaphoreType.DMA,
            pltpu.SemaphoreType.DMA,
        ],
    )


def _dense2_body(pa_ref, pb_ref, x_ref, w1_ref, b1_ref, h_ref):
    agg1 = jnp.concatenate([pa_ref[...], pb_ref[...]], axis=1)
    w1 = w1_ref[...]
    h_ref[...] = jnp.maximum(
        jnp.dot(agg1, w1[:, :32], preferred_element_type=jnp.float32)
        + b1_ref[0:1, :]
        + jnp.dot(x_ref[...], w1[:, 32:], preferred_element_type=jnp.float32),
        0.0)


def _final_body(pa_ref, pb_ref, h_ref, w2_ref, b2_ref, x1_ref, wlin_ref,
                blin_ref, out_ref, emb_ref):
    agg2 = pa_ref[...] + pb_ref[...]
    w2 = w2_ref[...]
    g = (jnp.dot(agg2, w2[:, :8], preferred_element_type=jnp.float32)
         + b2_ref[0:1, :8]
         + jnp.dot(h_ref[...], w2[:, 8:], preferred_element_type=jnp.float32))
    m = jnp.max(g, axis=1, keepdims=True)
    sft = g - m
    lse = jnp.log(jnp.sum(jnp.exp(sft), axis=1, keepdims=True))
    h2 = sft - lse
    emb_ref[...] = h2
    w = wlin_ref[...]
    cat = jnp.concatenate(
        [h2, x1_ref[...], jnp.zeros((h2.shape[0], 7), jnp.float32)], axis=1)
    o = jnp.dot(cat, w[:16, :1], preferred_element_type=jnp.float32) + blin_ref[0, 0]
    out_ref[...] = jnp.maximum(o, 0.0)


def kernel(x, edge_index, x1, edge_weight, W1_rel, b1_rel, W1_root,
           W2_rel, b2_rel, W2_root, W_lin, b_lin):
    f32 = jnp.float32
    src = edge_index[0]
    dst = edge_index[1]
    padn = _EP - _E
    fill = _N + (jnp.arange(padn, dtype=jnp.int32) % (_NP - _N))
    src_p = jnp.concatenate([src, fill]).reshape(_EP // _G, _G)
    dst_p = jnp.concatenate([dst, fill]).reshape(_EP // _G, _G)
    w_p = jnp.concatenate([edge_weight, jnp.zeros((padn,), f32)])
    x_pad = jnp.zeros((_NP, _D), f32).at[:_N].set(x)

    xc = jnp.concatenate([x_pad[:, :64], x_pad[:, 64:]], axis=0)
    p1 = _make_segsum_cols()(xc, src_p, dst_p, w_p)

    Wcat1 = jnp.concatenate([W1_rel, W1_root], axis=1)
    b1b = jnp.broadcast_to(b1_rel, (8, 32))
    h = pl.pallas_call(
        _dense2_body,
        out_shape=jax.ShapeDtypeStruct((_NP, 32), f32),
        grid=(10,),
        in_specs=[
            pl.BlockSpec((_NP // 10, 64), lambda i: (i, 0)),
            pl.BlockSpec((_NP // 10, 64), lambda i: (i + 10, 0)),
            pl.BlockSpec((_NP // 10, _D), lambda i: (i, 0)),
            pl.BlockSpec((_D, 64), lambda i: (0, 0)),
            pl.BlockSpec((8, 32), lambda i: (0, 0)),
        ],
        out_specs=pl.BlockSpec((_NP // 10, 32), lambda i: (i, 0)),
    )(p1, p1, x_pad, Wcat1, b1b)

    p2 = _make_segsum(32)(h, src_p, dst_p, w_p)

    W2cat = jnp.concatenate([W2_rel, W2_root], axis=1)
    b2b = jnp.broadcast_to(b2_rel, (8, 8))
    wlin_pad = jnp.zeros((16, 128), f32).at[:9, :1].set(W_lin)
    blin_pad = jnp.reshape(b_lin, (1, 1))
    x1_pad = jnp.zeros((_NP, 1), f32).at[:_N].set(x1)
    out, emb = pl.pallas_call(
        _final_body,
        out_shape=(
            jax.ShapeDtypeStruct((_NP, 1), f32),
            jax.ShapeDtypeStruct((_NP, 8), f32),
        ),
        grid=(10,),
        in_specs=[
            pl.BlockSpec((_NP // 10, 32), lambda i: (i, 0)),
            pl.BlockSpec((_NP // 10, 32), lambda i: (i + 10, 0)),
            pl.BlockSpec((_NP // 10, 32), lambda i: (i, 0)),
            pl.BlockSpec((32, 16), lambda i: (0, 0)),
            pl.BlockSpec((8, 8), lambda i: (0, 0)),
            pl.BlockSpec((_NP // 10, 1), lambda i: (i, 0)),
            pl.BlockSpec((16, 128), lambda i: (0, 0)),
            pl.BlockSpec((1, 1), lambda i: (0, 0), memory_space=pltpu.SMEM),
        ],
        out_specs=(
            pl.BlockSpec((_NP // 10, 1), lambda i: (i, 0)),
            pl.BlockSpec((_NP // 10, 8), lambda i: (i, 0)),
        ),
    )(p2, p2, h, W2cat, b2b, x1_pad, wlin_pad, blin_pad)
    return (out[:_N], emb[:_N])
